# Initial kernel scaffold; baseline (speedup 1.0000x reference)
import numpy as np
import jax
import jax.numpy as jnp
from jax.experimental import pallas as pl
from jax.experimental.pallas import tpu as pltpu

_K = 3
_EPS = 1e-5


def _round_up(n, m):
    return ((n + m - 1) // m) * m


def _acc_tree(terms):
    while len(terms) > 1:
        nxt = [terms[i] + terms[i + 1] for i in range(0, len(terms) - 1, 2)]
        if len(terms) % 2:
            nxt.append(terms[-1])
        terms = nxt
    return terms[0]


def _pool_mat(l_in, l_out):
    m = np.zeros((l_in, l_out), np.float32)
    for j in range(l_out):
        s = (j * l_in) // l_out
        e = -((-(j + 1) * l_in) // l_out)
        m[s:e, j] = 1.0 / (e - s)
    return m


def _make_body(cin, length):
    L = length

    def body(c_ref,
             w1_ref,
             w2_ref,
             w3_ref,
             x_ref,
             pw_ref,
             bm1_ref,
             wm2_ref,
             bm2_ref,
             o_ref):
        taps = [[], [], []]
        for ci in range(cin):
            xc = x_ref[:, ci * L:(ci + 1) * L]
            for k in range(_K):
                taps[k].append(w1_ref[ci * _K + k] * xc)
        y0 = _acc_tree(taps[0])
        y1 = _acc_tree(taps[1])
        y2 = _acc_tree(taps[2])
        h = y0 + pltpu.roll(y1, -1, 1) + pltpu.roll(y2, -2, 1)
        h = jnp.maximum(h + c_ref[0], 0.0)

        h2 = (w2_ref[0] * h + w2_ref[1] * pltpu.roll(h, -1, 1)
              + w2_ref[2] * pltpu.roll(h, -2, 1))
        h2 = jnp.maximum(h2 + c_ref[1], 0.0)
        h3 = (w3_ref[0] * h2 + w3_ref[1] * pltpu.roll(h2, -1, 1)
              + w3_ref[2] * pltpu.roll(h2, -2, 1))
        h3 = jnp.maximum(h3 + c_ref[2], 0.0)

        z = jnp.dot(h3, pw_ref[...], preferred_element_type=jnp.float32)
        z = jnp.maximum(z + bm1_ref[...], 0.0)
        o_ref[...] = (jnp.dot(z, wm2_ref[...], preferred_element_type=jnp.float32)
                      + bm2_ref[...])

    return body


def kernel(x, w1_full, b1, g1, beta1, mean1, var1,
           w2_full, b2, g2, beta2, mean2, var2,
           w3_full, b3, g3, beta3, mean3, var3,
           wm1, bm1, wm2, bm2, w1, w2, w3, c):
    B, Cin, L = x.shape
    hid_dim = wm1.shape[0]
    n_hidden = wm1.shape[1]
    out_dim = wm2.shape[1]
    L3 = L - 3 * (_K - 1)

    TB = min(128, _round_up(B, 8))
    B_pad = _round_up(B, TB)

    x2d = x.reshape(B, Cin * L).astype(jnp.float32)
    if B_pad != B:
        x2d = jnp.pad(x2d, ((0, B_pad - B), (0, 0)))

    pool = jnp.asarray(_pool_mat(L3, hid_dim))
    pw = pool @ wm1
    pw_pad = jnp.zeros((L, n_hidden), jnp.float32).at[:L3].set(pw)

    flops = 2 * B_pad * (Cin * _K * L + 2 * _K * L
                         + L * n_hidden + n_hidden * out_dim)
    bytes_accessed = 4 * (x2d.size + pw_pad.size + n_hidden
                          + n_hidden * out_dim + out_dim + B_pad * out_dim
                          + Cin * _K + 2 * _K + 3)

    out = pl.pallas_call(
        _make_body(Cin, L),
        out_shape=jax.ShapeDtypeStruct((B_pad, out_dim), jnp.float32),
        grid_spec=pltpu.PrefetchScalarGridSpec(
            num_scalar_prefetch=4,
            grid=(B_pad // TB,),
            in_specs=[
                pl.BlockSpec((TB, Cin * L), lambda b, *_: (b, 0)),
                pl.BlockSpec((L, n_hidden), lambda b, *_: (0, 0)),
                pl.BlockSpec((1, n_hidden), lambda b, *_: (0, 0)),
                pl.BlockSpec((n_hidden, out_dim), lambda b, *_: (0, 0)),
                pl.BlockSpec((1, out_dim), lambda b, *_: (0, 0)),
            ],
            out_specs=pl.BlockSpec((TB, out_dim), lambda b, *_: (b, 0)),
        ),
        compiler_params=pltpu.CompilerParams(
            dimension_semantics=("parallel",),
            vmem_limit_bytes=64 * 1024 * 1024,
        ),
        cost_estimate=pl.CostEstimate(flops=flops, transcendentals=0,
                                      bytes_accessed=bytes_accessed),
    )(c, w1, w2, w3, x2d, pw_pad, bm1, wm2, bm2)

    return out[:B]


# speedup vs baseline: 1.3348x; 1.3348x over previous
import numpy as np
import jax
import jax.numpy as jnp
from jax.experimental import pallas as pl
from jax.experimental.pallas import tpu as pltpu

_K = 3
_EPS = 1e-5


def _round_up(n, m):
    return ((n + m - 1) // m) * m


def _acc_tree(terms):
    while len(terms) > 1:
        nxt = [terms[i] + terms[i + 1] for i in range(0, len(terms) - 1, 2)]
        if len(terms) % 2:
            nxt.append(terms[-1])
        terms = nxt
    return terms[0]


def _pool_mat(l_in, l_out):
    m = np.zeros((l_in, l_out), np.float32)
    for j in range(l_out):
        s = (j * l_in) // l_out
        e = -((-(j + 1) * l_in) // l_out)
        m[s:e, j] = 1.0 / (e - s)
    return m


def _make_body(cin, length):
    L = length

    def body(c_ref,
             w1_ref,
             w2_ref,
             w3_ref,
             x_ref,
             pw_ref,
             bm1_ref,
             wm2_ref,
             bm2_ref,
             o_ref):
        taps = [[], [], []]
        for ci in range(cin):
            xc = x_ref[:, ci * L:(ci + 1) * L]
            for k in range(_K):
                taps[k].append(w1_ref[ci * _K + k] * xc)
        y0 = _acc_tree(taps[0])
        y1 = _acc_tree(taps[1])
        y2 = _acc_tree(taps[2])
        h = y0 + pltpu.roll(y1, L - 1, 1) + pltpu.roll(y2, L - 2, 1)
        h = jnp.maximum(h + c_ref[0], 0.0)

        h2 = (w2_ref[0] * h + w2_ref[1] * pltpu.roll(h, L - 1, 1)
              + w2_ref[2] * pltpu.roll(h, L - 2, 1))
        h2 = jnp.maximum(h2 + c_ref[1], 0.0)
        h3 = (w3_ref[0] * h2 + w3_ref[1] * pltpu.roll(h2, L - 1, 1)
              + w3_ref[2] * pltpu.roll(h2, L - 2, 1))
        h3 = jnp.maximum(h3 + c_ref[2], 0.0)

        z = jnp.dot(h3, pw_ref[...], preferred_element_type=jnp.float32)
        z = jnp.maximum(z + bm1_ref[...], 0.0)
        o_ref[...] = (jnp.dot(z, wm2_ref[...], preferred_element_type=jnp.float32)
                      + bm2_ref[...])

    return body


def kernel(x, w1_full, b1, g1, beta1, mean1, var1,
           w2_full, b2, g2, beta2, mean2, var2,
           w3_full, b3, g3, beta3, mean3, var3,
           wm1, bm1, wm2, bm2, w1, w2, w3, c):
    B, Cin, L = x.shape
    hid_dim = wm1.shape[0]
    n_hidden = wm1.shape[1]
    out_dim = wm2.shape[1]
    L3 = L - 3 * (_K - 1)

    TB = min(128, _round_up(B, 8))
    B_pad = _round_up(B, TB)

    x2d = x.reshape(B, Cin * L).astype(jnp.float32)
    if B_pad != B:
        x2d = jnp.pad(x2d, ((0, B_pad - B), (0, 0)))

    pool = jnp.asarray(_pool_mat(L3, hid_dim))
    pw = pool @ wm1
    pw_pad = jnp.zeros((L, n_hidden), jnp.float32).at[:L3].set(pw)

    flops = 2 * B_pad * (Cin * _K * L + 2 * _K * L
                         + L * n_hidden + n_hidden * out_dim)
    bytes_accessed = 4 * (x2d.size + pw_pad.size + n_hidden
                          + n_hidden * out_dim + out_dim + B_pad * out_dim
                          + Cin * _K + 2 * _K + 3)

    out = pl.pallas_call(
        _make_body(Cin, L),
        out_shape=jax.ShapeDtypeStruct((B_pad, out_dim), jnp.float32),
        grid_spec=pltpu.PrefetchScalarGridSpec(
            num_scalar_prefetch=4,
            grid=(B_pad // TB,),
            in_specs=[
                pl.BlockSpec((TB, Cin * L), lambda b, *_: (b, 0)),
                pl.BlockSpec((L, n_hidden), lambda b, *_: (0, 0)),
                pl.BlockSpec((1, n_hidden), lambda b, *_: (0, 0)),
                pl.BlockSpec((n_hidden, out_dim), lambda b, *_: (0, 0)),
                pl.BlockSpec((1, out_dim), lambda b, *_: (0, 0)),
            ],
            out_specs=pl.BlockSpec((TB, out_dim), lambda b, *_: (b, 0)),
        ),
        compiler_params=pltpu.CompilerParams(
            dimension_semantics=("parallel",),
            vmem_limit_bytes=64 * 1024 * 1024,
        ),
        cost_estimate=pl.CostEstimate(flops=flops, transcendentals=0,
                                      bytes_accessed=bytes_accessed),
    )(c, w1, w2, w3, x2d, pw_pad, bm1, wm2, bm2)

    return out[:B]


# speedup vs baseline: 3.0883x; 2.3137x over previous
import numpy as np
import jax
import jax.numpy as jnp
from jax.experimental import pallas as pl
from jax.experimental.pallas import tpu as pltpu

_K = 3
_EPS = 1e-5


def _round_up(n, m):
    return ((n + m - 1) // m) * m


def _pool_mat(l_in, l_out):
    m = np.zeros((l_in, l_out), np.float32)
    for j in range(l_out):
        s = (j * l_in) // l_out
        e = -((-(j + 1) * l_in) // l_out)
        m[s:e, j] = 1.0 / (e - s)
    return m


def _make_body(cin, length, tb):
    L = length
    TB = tb

    def body(c_ref,
             w2_ref,
             w3_ref,
             x_ref,
             a_ref,
             pw_ref,
             bm1_ref,
             wm2_ref,
             bm2_ref,
             o_ref):
        x2 = x_ref[...].reshape(TB * cin, L)
        y = jnp.dot(a_ref[...], x2, preferred_element_type=jnp.float32)
        h = (y[0:TB] + pltpu.roll(y[TB:2 * TB], L - 1, 1)
             + pltpu.roll(y[2 * TB:3 * TB], L - 2, 1))
        h = jnp.maximum(h + c_ref[0], 0.0)

        h2 = (w2_ref[0] * h + w2_ref[1] * pltpu.roll(h, L - 1, 1)
              + w2_ref[2] * pltpu.roll(h, L - 2, 1))
        h2 = jnp.maximum(h2 + c_ref[1], 0.0)
        h3 = (w3_ref[0] * h2 + w3_ref[1] * pltpu.roll(h2, L - 1, 1)
              + w3_ref[2] * pltpu.roll(h2, L - 2, 1))
        h3 = jnp.maximum(h3 + c_ref[2], 0.0)

        z = jnp.dot(h3, pw_ref[...], preferred_element_type=jnp.float32)
        z = jnp.maximum(z + bm1_ref[...], 0.0)
        o_ref[...] = (jnp.dot(z, wm2_ref[...], preferred_element_type=jnp.float32)
                      + bm2_ref[...])

    return body


def kernel(x, w1_full, b1, g1, beta1, mean1, var1,
           w2_full, b2, g2, beta2, mean2, var2,
           w3_full, b3, g3, beta3, mean3, var3,
           wm1, bm1, wm2, bm2, w1, w2, w3, c):
    B, Cin, L = x.shape
    hid_dim = wm1.shape[0]
    n_hidden = wm1.shape[1]
    out_dim = wm2.shape[1]
    L3 = L - 3 * (_K - 1)

    TB = min(128, _round_up(B, 8))
    B_pad = _round_up(B, TB)

    x3d = x.astype(jnp.float32)
    if B_pad != B:
        x3d = jnp.pad(x3d, ((0, B_pad - B), (0, 0), (0, 0)))

    w_ck = w1.reshape(Cin, _K)
    amat = jnp.einsum('ck,jJ->kjJc', w_ck, jnp.eye(TB, dtype=jnp.float32))
    amat = amat.reshape(_K * TB, TB * Cin)

    pool = jnp.asarray(_pool_mat(L3, hid_dim))
    pw = pool @ wm1
    pw_pad = jnp.zeros((L, n_hidden), jnp.float32).at[:L3].set(pw)

    flops = 2 * B_pad * (_K * TB * Cin * L // TB * TB + 2 * _K * L
                         + L * n_hidden + n_hidden * out_dim)
    bytes_accessed = 4 * (x3d.size + amat.size + pw_pad.size + n_hidden
                          + n_hidden * out_dim + out_dim + B_pad * out_dim
                          + 2 * _K + 3)

    out = pl.pallas_call(
        _make_body(Cin, L, TB),
        out_shape=jax.ShapeDtypeStruct((B_pad, out_dim), jnp.float32),
        grid_spec=pltpu.PrefetchScalarGridSpec(
            num_scalar_prefetch=3,
            grid=(B_pad // TB,),
            in_specs=[
                pl.BlockSpec((TB, Cin, L), lambda b, *_: (b, 0, 0)),
                pl.BlockSpec((_K * TB, TB * Cin), lambda b, *_: (0, 0)),
                pl.BlockSpec((L, n_hidden), lambda b, *_: (0, 0)),
                pl.BlockSpec((1, n_hidden), lambda b, *_: (0, 0)),
                pl.BlockSpec((n_hidden, out_dim), lambda b, *_: (0, 0)),
                pl.BlockSpec((1, out_dim), lambda b, *_: (0, 0)),
            ],
            out_specs=pl.BlockSpec((TB, out_dim), lambda b, *_: (b, 0)),
        ),
        compiler_params=pltpu.CompilerParams(
            dimension_semantics=("parallel",),
            vmem_limit_bytes=64 * 1024 * 1024,
        ),
        cost_estimate=pl.CostEstimate(flops=flops, transcendentals=0,
                                      bytes_accessed=bytes_accessed),
    )(c, w2, w3, x3d, amat, pw_pad, bm1, wm2, bm2)

    return out[:B]
